# baseline (speedup 1.0000x reference)
import functools

import jax
import jax.numpy as jnp
from jax import lax
from jax.experimental import pallas as pl
from jax.experimental.pallas import tpu as pltpu
from jax.experimental.pallas import tpu_sc as plsc

NC = 2
NS = 16
NW = NC * NS
B = 16384
D = 32
BPW = B // NW
NCOLS = 7813
SBASE = NCOLS // NW
SREM = NCOLS % NW
S = 16
NB = B // 16

_mesh = plsc.VectorSubcoreMesh(core_axis_name="c", subcore_axis_name="s")
_params = pltpu.CompilerParams(
    needs_layout_passes=False, use_tc_tiling_on_sc=True)

def _strip_bounds(wid):
    rb_lo = wid * SBASE + jnp.minimum(wid, SREM)
    ncols = SBASE + jnp.where(wid < SREM, 1, 0)
    return rb_lo, ncols


def _bin_indices(idx_all, bins_r, bins_k, counts, rb_lo, rb_hi, overflow_fn):
    lane = jnp.arange(16, dtype=jnp.int32)

    def batch(c, _):
        r16 = idx_all[pl.ds(c * 16, 16)]
        kv = c * 16 + lane
        rb = r16 >> 7
        m = (rb >= rb_lo) & (rb < rb_hi)
        npc = plsc.all_reduce_population_count(m)

        @pl.when(npc[0] > 0)
        def _():
            rbl = jnp.where(m, rb - rb_lo, 0)
            dup, last = plsc.scan_count(rbl, mask=m)
            old = plsc.load_gather(counts, [rbl])
            slot = old + dup
            ok = m & (slot < S)
            pos = rbl * S + jnp.minimum(slot, S - 1)
            plsc.store_scatter(bins_r, [pos], r16, mask=ok)
            plsc.store_scatter(bins_k, [pos], kv, mask=ok)
            plsc.store_scatter(counts, [rbl], slot + 1, mask=m & last)
            ovf = m & (slot >= S)
            novf = plsc.all_reduce_population_count(ovf)

            @pl.when(novf[0] > 0)
            def _():
                ovfi = ovf.astype(jnp.int32)
                for l in range(16):
                    @pl.when(ovfi[l] > 0)
                    def _(l=l):
                        overflow_fn(r16[l], kv[l])
        return 0

    lax.fori_loop(0, NB, batch, 0)


def _init_bins(bins_r, counts):
    neg = jnp.full((16,), -1, jnp.int32)
    zero = jnp.zeros((16,), jnp.int32)

    def ib(i, _):
        bins_r[pl.ds(i * 16, 16)] = neg
        return 0
    lax.fori_loop(0, (SBASE + 1) * S // 16, ib, 0)
    for i in range(16):
        counts[pl.ds(i * 16, 16)] = zero


def _extract_row(blk, col):
    lane = jnp.arange(16, dtype=jnp.int32)
    c = jnp.full((16,), 0, jnp.int32) + col
    v0 = plsc.load_gather(blk, [lane, c])
    v1 = plsc.load_gather(blk, [lane + 16, c])
    return v0, v1


@functools.partial(
    pl.kernel,
    mesh=_mesh,
    out_type=jax.ShapeDtypeStruct((B * D,), jnp.float32),
    scratch_types=[
        pltpu.VMEM((B,), jnp.int32),
        pltpu.VMEM(((SBASE + 1) * S,), jnp.int32),
        pltpu.VMEM(((SBASE + 1) * S,), jnp.int32),
        pltpu.VMEM((256,), jnp.int32),
        pltpu.VMEM((2, D, 128), jnp.float32),
        pltpu.VMEM((D, 128), jnp.float32),
        pltpu.VMEM((2, 16, D), jnp.float32),
        pltpu.SemaphoreType.DMA,
        pltpu.SemaphoreType.DMA,
        pltpu.SemaphoreType.DMA,
    ],
    compiler_params=_params,
)
def _k1_user(user_hbm, ut_hbm, u1d_hbm,
             idx_all, bins_r, bins_k, counts, colblk, ovrblk, rowbuf,
             sem_c, sem_o, sem_w):
    wid = lax.axis_index("s") * NC + lax.axis_index("c")
    rb_lo, ncols = _strip_bounds(wid)
    rb_hi = rb_lo + ncols

    pltpu.sync_copy(user_hbm, idx_all)
    _init_bins(bins_r, counts)

    def overflow(r, k):
        cb = pl.multiple_of((r >> 7) << 7, 128)
        pltpu.async_copy(ut_hbm.at[:, pl.ds(cb, 128)], ovrblk, sem_o).wait()
        v0, v1 = _extract_row(ovrblk, r & 127)
        rowbuf[0, 0, pl.ds(0, 16)] = v0
        rowbuf[0, 0, pl.ds(16, 16)] = v1
        pltpu.async_copy(
            rowbuf.at[0, 0], u1d_hbm.at[pl.ds(pl.multiple_of(k * D, 8), D)],
            sem_o).wait()

    _bin_indices(idx_all, bins_r, bins_k, counts, rb_lo, rb_hi, overflow)

    lane = jnp.arange(16, dtype=jnp.int32)

    def fetch(c, buf):
        cb = pl.multiple_of((rb_lo + c) * 128, 128)
        pltpu.async_copy(ut_hbm.at[:, pl.ds(cb, 128)],
                         colblk.at[buf], sem_c)

    fetch(0, 0)

    def col_body(c, _):
        buf = c & 1

        @pl.when(c + 1 < ncols)
        def _():
            fetch(c + 1, 1 - buf)

        pltpu.make_async_copy(ut_hbm.at[:, pl.ds(0, 128)],
                              colblk.at[buf], sem_c).wait()

        rvec = bins_r[pl.ds(c * S, 16)]
        valid = rvec >= 0
        validi = valid.astype(jnp.int32)
        nv = plsc.all_reduce_population_count(valid)

        @pl.when(nv[0] > 0)
        def _():
            kvec = bins_k[pl.ds(c * S, 16)]
            colv = jnp.where(valid, rvec & 127, 0)
            bsp = jnp.zeros((16,), jnp.int32) + buf
            for d in range(D):
                dv = jnp.full((16,), d, jnp.int32)
                val = plsc.load_gather(colblk, [bsp, dv, colv])
                plsc.store_scatter(rowbuf, [bsp, lane, dv], val, mask=valid)
            for l in range(S):
                @pl.when(validi[l] > 0)
                def _(l=l):
                    kk = pl.multiple_of(kvec[l] * D, 8)
                    pltpu.async_copy(
                        rowbuf.at[buf, l], u1d_hbm.at[pl.ds(kk, D)], sem_w)
            for l in range(S):
                @pl.when(validi[l] > 0)
                def _(l=l):
                    pltpu.make_async_copy(
                        rowbuf.at[buf, l], u1d_hbm.at[pl.ds(0, D)],
                        sem_w).wait()
        return 0

    lax.fori_loop(0, ncols, col_body, 0)


@functools.partial(
    pl.kernel,
    mesh=_mesh,
    out_type=jax.ShapeDtypeStruct((B * 16,), jnp.float32),
    scratch_types=[
        pltpu.VMEM((B,), jnp.int32),
        pltpu.VMEM(((SBASE + 1) * S,), jnp.int32),
        pltpu.VMEM(((SBASE + 1) * S,), jnp.int32),
        pltpu.VMEM((256,), jnp.int32),
        pltpu.VMEM((2, D, 128), jnp.float32),
        pltpu.VMEM((D, 128), jnp.float32),
        pltpu.VMEM((2, 16, D), jnp.float32),
        pltpu.VMEM((2, 16, 16), jnp.float32),
        pltpu.VMEM((48,), jnp.float32),
        pltpu.SemaphoreType.DMA,
        pltpu.SemaphoreType.DMA,
        pltpu.SemaphoreType.DMA,
        pltpu.SemaphoreType.DMA,
    ],
    compiler_params=_params,
)
def _k2_item(item_hbm, it_hbm, u1d_hbm, wb_hbm, res_hbm,
             idx_all, bins_r, bins_k, counts, colblk, ovrblk, ubuf, resbuf,
             wb_v, sem_c, sem_o, sem_u, sem_w):
    wid = lax.axis_index("s") * NC + lax.axis_index("c")
    rb_lo, ncols = _strip_bounds(wid)
    rb_hi = rb_lo + ncols

    pltpu.sync_copy(item_hbm, idx_all)
    pltpu.sync_copy(wb_hbm, wb_v)
    _init_bins(bins_r, counts)

    lane = jnp.arange(16, dtype=jnp.int32)
    w_lo = wb_v[pl.ds(0, 16)]
    w_hi = wb_v[pl.ds(16, 16)]
    bv = wb_v[pl.ds(32, 16)]

    def overflow(r, k):
        cb = pl.multiple_of((r >> 7) << 7, 128)
        pltpu.async_copy(it_hbm.at[:, pl.ds(cb, 128)], ovrblk, sem_o).wait()
        i0, i1 = _extract_row(ovrblk, r & 127)
        pltpu.async_copy(
            u1d_hbm.at[pl.ds(pl.multiple_of(k * D, 8), D)], ubuf.at[0, 0],
            sem_o).wait()
        u0 = ubuf[0, 0, pl.ds(0, 16)]
        u1 = ubuf[0, 0, pl.ds(16, 16)]
        s = jnp.sum(u0 * i0 * w_lo + u1 * i1 * w_hi) + bv[0]
        sv = jnp.zeros((16,), jnp.float32) + s
        resbuf[0, 0, pl.ds(0, 16)] = 1.0 / (1.0 + jnp.exp(-sv))
        pltpu.async_copy(
            resbuf.at[0, 0], res_hbm.at[pl.ds(pl.multiple_of(k * 16, 8), 16)],
            sem_o).wait()

    _bin_indices(idx_all, bins_r, bins_k, counts, rb_lo, rb_hi, overflow)

    def fetch(c, buf):
        cb = pl.multiple_of((rb_lo + c) * 128, 128)
        pltpu.async_copy(it_hbm.at[:, pl.ds(cb, 128)],
                         colblk.at[buf], sem_c)
        kvec = bins_k[pl.ds(c * S, 16)]
        rvec = bins_r[pl.ds(c * S, 16)]
        validi = (rvec >= 0).astype(jnp.int32)
        for l in range(S):
            @pl.when(validi[l] > 0)
            def _(l=l):
                kk = pl.multiple_of(kvec[l] * D, 8)
                pltpu.async_copy(
                    u1d_hbm.at[pl.ds(kk, D)], ubuf.at[buf, l], sem_u)

    fetch(0, 0)

    def col_body(c, _):
        buf = c & 1

        @pl.when(c + 1 < ncols)
        def _():
            fetch(c + 1, 1 - buf)

        pltpu.make_async_copy(it_hbm.at[:, pl.ds(0, 128)],
                              colblk.at[buf], sem_c).wait()

        rvec = bins_r[pl.ds(c * S, 16)]
        valid = rvec >= 0
        validi = valid.astype(jnp.int32)
        nv = plsc.all_reduce_population_count(valid)

        @pl.when(nv[0] > 0)
        def _():
            kvec = bins_k[pl.ds(c * S, 16)]
            for l in range(S):
                @pl.when(validi[l] > 0)
                def _(l=l):
                    pltpu.make_async_copy(
                        u1d_hbm.at[pl.ds(0, D)], ubuf.at[buf, l], sem_u).wait()
            colv = jnp.where(valid, rvec & 127, 0)
            bsp = jnp.zeros((16,), jnp.int32) + buf
            accs = [jnp.zeros((16,), jnp.float32) for _ in range(4)]
            for d in range(D):
                dv = jnp.full((16,), d, jnp.int32)
                ivd = plsc.load_gather(colblk, [bsp, dv, colv])
                uvd = plsc.load_gather(ubuf, [bsp, lane, dv])
                w_d = (w_lo if d < 16 else w_hi)[d % 16]
                accs[d % 4] = accs[d % 4] + (uvd * ivd) * w_d
            s = (accs[0] + accs[1]) + (accs[2] + accs[3]) + bv
            sig = 1.0 / (1.0 + jnp.exp(-s))
            for l in range(S):
                @pl.when(validi[l] > 0)
                def _(l=l):
                    resbuf[buf, l, pl.ds(0, 16)] = (
                        jnp.zeros((16,), jnp.float32) + sig[l])
                    kk = pl.multiple_of(kvec[l] * 16, 8)
                    pltpu.async_copy(
                        resbuf.at[buf, l], res_hbm.at[pl.ds(kk, 16)], sem_w)
            for l in range(S):
                @pl.when(validi[l] > 0)
                def _(l=l):
                    pltpu.make_async_copy(
                        resbuf.at[buf, l], res_hbm.at[pl.ds(0, 16)],
                        sem_w).wait()
        return 0

    lax.fori_loop(0, ncols, col_body, 0)


@functools.partial(
    pl.kernel,
    mesh=_mesh,
    out_type=jax.ShapeDtypeStruct((B,), jnp.float32),
    scratch_types=[
        pltpu.VMEM((BPW * 16,), jnp.float32),
        pltpu.VMEM((BPW,), jnp.float32),
    ],
    compiler_params=_params,
)
def _k3_gather(res_hbm, out_hbm, resv, out_v):
    wid = lax.axis_index("s") * NC + lax.axis_index("c")
    base = wid * BPW
    pltpu.sync_copy(res_hbm.at[pl.ds(base * 16, BPW * 16)], resv)
    lane = jnp.arange(16, dtype=jnp.int32)
    for g in range(BPW // 16):
        idx = g * 256 + lane * 16
        out_v[pl.ds(g * 16, 16)] = plsc.load_gather(resv, [idx])
    pltpu.sync_copy(out_v, out_hbm.at[pl.ds(base, BPW)])


def kernel(user, item, user_table, item_table, W, b):
    wb = jnp.concatenate(
        [W.reshape(-1), jnp.broadcast_to(b.reshape(-1), (16,))]).astype(jnp.float32)
    u1d = _k1_user(user.astype(jnp.int32), user_table.T)
    res = _k2_item(item.astype(jnp.int32), item_table.T, u1d, wb)
    return _k3_gather(res)


# speedup vs baseline: 1.7690x; 1.7690x over previous
import functools

import jax
import jax.numpy as jnp
from jax import lax
from jax.experimental import pallas as pl
from jax.experimental.pallas import tpu as pltpu
from jax.experimental.pallas import tpu_sc as plsc

NC = 2
NS = 16
NW = NC * NS
B = 16384
D = 32
BPW = B // NW
NPAIR = BPW // 16

_mesh = plsc.VectorSubcoreMesh(core_axis_name="c", subcore_axis_name="s")


@functools.partial(
    pl.kernel,
    mesh=_mesh,
    out_type=jax.ShapeDtypeStruct((B,), jnp.float32),
    scratch_types=[
        pltpu.VMEM((BPW,), jnp.int32),
        pltpu.VMEM((BPW,), jnp.int32),
        pltpu.VMEM((8, D, 128), jnp.float32),
        pltpu.VMEM((8, D, 128), jnp.float32),
        pltpu.VMEM((48,), jnp.float32),
        pltpu.VMEM((BPW,), jnp.float32),
        pltpu.SemaphoreType.DMA,
        pltpu.SemaphoreType.DMA,
    ],
    compiler_params=pltpu.CompilerParams(
        needs_layout_passes=False, use_tc_tiling_on_sc=True),
)
def _gmf_sc(user_hbm, item_hbm, ut_hbm, it_hbm, wb_hbm, out_hbm,
            idx_uv, idx_iv, blk_u, blk_i, wb_v, out_v, sem_u, sem_i):
    wid = lax.axis_index("s") * NC + lax.axis_index("c")
    base = wid * BPW

    pltpu.sync_copy(wb_hbm, wb_v)
    pltpu.sync_copy(user_hbm.at[pl.ds(base, BPW)], idx_uv)
    pltpu.sync_copy(item_hbm.at[pl.ds(base, BPW)], idx_iv)

    lane = jnp.arange(16, dtype=jnp.int32)
    half = lane < 8
    jv = lane & 7
    bv = wb_v[pl.ds(32, 16)]
    w_lo = wb_v[pl.ds(0, 16)]
    w_hi = wb_v[pl.ds(16, 16)]

    def pair_body(p, _):
        e0 = p * 16
        ru16 = idx_uv[pl.ds(e0, 16)]
        ri16 = idx_iv[pl.ds(e0, 16)]
        cbu = (ru16 >> 7) << 7
        cbi = (ri16 >> 7) << 7
        col_u = ru16 & 127
        col_i = ri16 & 127
        accs = [None, None]
        for h in range(2):
            copies = []
            for j in range(8):
                cu = pl.multiple_of(cbu[h * 8 + j], 128)
                copies.append(pltpu.async_copy(
                    ut_hbm.at[:, pl.ds(cu, 128)], blk_u.at[j], sem_u))
                ci = pl.multiple_of(cbi[h * 8 + j], 128)
                copies.append(pltpu.async_copy(
                    it_hbm.at[:, pl.ds(ci, 128)], blk_i.at[j], sem_i))
            for cp in copies:
                cp.wait()
            pacc = [jnp.zeros((16,), jnp.float32) for _ in range(4)]
            for d in range(D):
                dv = jnp.full((16,), d, jnp.int32)
                u = plsc.load_gather(blk_u, [jv, dv, col_u])
                iv = plsc.load_gather(blk_i, [jv, dv, col_i])
                w_d = (w_lo if d < 16 else w_hi)[d % 16]
                pacc[d % 4] = pacc[d % 4] + (u * iv) * w_d
            accs[h] = (pacc[0] + pacc[1]) + (pacc[2] + pacc[3])
        s = jnp.where(half, accs[0], accs[1]) + bv
        out_v[pl.ds(e0, 16)] = 1.0 / (1.0 + jnp.exp(-s))
        return 0

    lax.fori_loop(0, NPAIR, pair_body, 0)

    pltpu.sync_copy(out_v, out_hbm.at[pl.ds(base, BPW)])


def kernel(user, item, user_table, item_table, W, b):
    wb = jnp.concatenate(
        [W.reshape(-1), jnp.broadcast_to(b.reshape(-1), (16,))]).astype(jnp.float32)
    return _gmf_sc(user.astype(jnp.int32), item.astype(jnp.int32),
                   user_table.T, item_table.T, wb)
